# baseline (speedup 1.0000x reference)
import functools

import jax
import jax.numpy as jnp
from jax import lax
from jax.experimental import pallas as pl
from jax.experimental.pallas import tpu as pltpu
from jax.experimental.pallas import tpu_sc as plsc

PW = 16


def _sc_gather(xflat, ppad, xidx, pidx, rows_per_worker):
    rows, c = xidx.shape[0], xflat.shape[1]
    nw = 32
    assert rows == nw * rows_per_worker
    chunk = 128
    nfull = rows_per_worker // chunk
    tail = rows_per_worker - nfull * chunk
    assert nfull % 2 == 0 and tail % 8 == 0 and tail < chunk

    mesh = plsc.VectorSubcoreMesh(core_axis_name="c", subcore_axis_name="s")

    @functools.partial(
        pl.kernel,
        out_type=[
            jax.ShapeDtypeStruct((rows, c), jnp.bfloat16),
            jax.ShapeDtypeStruct((rows, PW), jnp.float32),
        ],
        mesh=mesh,
        compiler_params=pltpu.CompilerParams(use_tc_tiling_on_sc=False),
        scratch_types=[
            pltpu.VMEM((rows_per_worker,), jnp.int32),
            pltpu.VMEM((rows_per_worker,), jnp.int32),
            pltpu.VMEM((chunk, c), jnp.bfloat16),
            pltpu.VMEM((chunk, c), jnp.bfloat16),
            pltpu.VMEM((chunk, PW), jnp.float32),
            pltpu.VMEM((chunk, PW), jnp.float32),
            pltpu.SemaphoreType.DMA,
            pltpu.SemaphoreType.DMA,
            pltpu.SemaphoreType.DMA,
            pltpu.SemaphoreType.DMA,
        ],
    )
    def k(xflat_hbm, ppad_hbm, xidx_hbm, pidx_hbm, nbrx_hbm, nbrp_hbm,
          xidx_v, pidx_v, xr0, xr1, pr0, pr1, sx0, sx1, sp0, sp1):
        wid = lax.axis_index("s") * 2 + lax.axis_index("c")
        base = wid * rows_per_worker
        pltpu.sync_copy(xidx_hbm.at[pl.ds(base, rows_per_worker)], xidx_v)
        pltpu.sync_copy(pidx_hbm.at[pl.ds(base, rows_per_worker)], pidx_v)

        def start(g, xr, pr, sx, sp, nrows=chunk):
            off = pl.multiple_of(g * chunk, chunk)
            pltpu.async_copy(
                xflat_hbm.at[xidx_v.at[pl.ds(off, nrows)]],
                xr.at[pl.ds(0, nrows)], sx)
            pltpu.async_copy(
                ppad_hbm.at[pidx_v.at[pl.ds(off, nrows)]],
                pr.at[pl.ds(0, nrows)], sp)

        def drain(xr, pr, sx, sp, nrows=chunk):
            pltpu.make_async_copy(
                xflat_hbm.at[pl.ds(0, nrows)], xr.at[pl.ds(0, nrows)],
                sx).wait()
            pltpu.make_async_copy(
                ppad_hbm.at[pl.ds(0, nrows)], pr.at[pl.ds(0, nrows)],
                sp).wait()

        def write(g, xr, pr, nrows=chunk):
            off = pl.multiple_of(g * chunk, chunk)
            pltpu.sync_copy(xr.at[pl.ds(0, nrows)],
                            nbrx_hbm.at[pl.ds(base + off, nrows)])
            pltpu.sync_copy(pr.at[pl.ds(0, nrows)],
                            nbrp_hbm.at[pl.ds(base + off, nrows)])

        start(0, xr0, pr0, sx0, sp0)

        def body(go, carry):
            g0 = pl.multiple_of(go * 2, 2)
            start(g0 + 1, xr1, pr1, sx1, sp1)
            drain(xr0, pr0, sx0, sp0)
            write(g0, xr0, pr0)
            start(g0 + 2, xr0, pr0, sx0, sp0)
            drain(xr1, pr1, sx1, sp1)
            write(g0 + 1, xr1, pr1)
            return carry

        lax.fori_loop(0, nfull // 2 - 1, body, 0)
        g0 = nfull - 2
        start(g0 + 1, xr1, pr1, sx1, sp1)
        drain(xr0, pr0, sx0, sp0)
        write(g0, xr0, pr0)
        if tail:
            start(nfull, xr0, pr0, sx0, sp0, nrows=tail)
        drain(xr1, pr1, sx1, sp1)
        write(g0 + 1, xr1, pr1)
        if tail:
            drain(xr0, pr0, sx0, sp0, nrows=tail)
            write(nfull, xr0, pr0, nrows=tail)

    return k(xflat, ppad, xidx, pidx)


def _tc_body(nbrp_ref, nbrx_ref, pp_ref, a_ref, w1p_ref, g1_ref,
             be1_ref, w2_ref, b2_ref, g2_ref, be2_ref, out_ref):
    b, kk, tn, c = nbrx_ref.shape
    hid = w1p_ref.shape[1]
    out = w2_ref.shape[0]

    pp = pp_ref[...].reshape(b * tn, PW)
    hi = nbrp_ref[:, 0].reshape(b * tn, c).astype(jnp.bfloat16)
    lo = nbrp_ref[:, 1].reshape(b * tn, c).astype(jnp.bfloat16)
    h = (jnp.dot(pp, a_ref[...], preferred_element_type=jnp.float32)
         - (jnp.dot(hi, w1p_ref[0:c], preferred_element_type=jnp.float32)
            + jnp.dot(lo, w1p_ref[c:2 * c],
                      preferred_element_type=jnp.float32)))
    h3 = h.reshape(b, tn, hid)
    m1 = jnp.mean(h3, axis=(0, 2), keepdims=True)
    q1 = jnp.mean(h3 * h3, axis=(0, 2), keepdims=True)
    rs1 = lax.rsqrt(q1 - m1 * m1 + 1e-5)
    sc1 = rs1 * g1_ref[...][None]
    sh1 = be1_ref[...][None] - m1 * sc1
    hr = jnp.maximum(h3 * sc1 + sh1, 0.0)
    hrb = hr.astype(jnp.bfloat16).reshape(b * tn, hid)

    o = lax.dot_general(hrb, w2_ref[...], (((1,), (1,)), ((), ())),
                        preferred_element_type=jnp.float32) + b2_ref[...]
    o3 = o.reshape(b, tn, out)
    m2 = jnp.mean(o3, axis=(0, 2), keepdims=True)
    q2 = jnp.mean(o3 * o3, axis=(0, 2), keepdims=True)
    rs2 = lax.rsqrt(q2 - m2 * m2 + 1e-5)
    sc2 = rs2 * g2_ref[...][None]
    sh2 = be2_ref[...][None] - m2 * sc2

    acc = jnp.zeros((b, tn, c), jnp.float32)
    for j in range(kk):
        yj = jnp.maximum(o3[:, :, j * c:(j + 1) * c] * sc2 + sh2, 0.0)
        acc = acc + yj * nbrx_ref[:, j].astype(jnp.float32)
    out_ref[...] = acc


def _tc_mlp(nbrp4, nbrx4, ppad3, a, w1p, g1c, be1c, w2, b2r, g2c, be2c, tn):
    b, kk, n, c = nbrx4.shape
    hid = w1p.shape[1]
    out = w2.shape[0]
    grid = (n // tn,)
    return pl.pallas_call(
        _tc_body,
        grid=grid,
        in_specs=[
            pl.BlockSpec((b, 2, tn, c), lambda i: (0, 0, i, 0)),
            pl.BlockSpec((b, kk, tn, c), lambda i: (0, 0, i, 0)),
            pl.BlockSpec((b, tn, PW), lambda i: (0, i, 0)),
            pl.BlockSpec((PW, hid), lambda i: (0, 0)),
            pl.BlockSpec((2 * c, hid), lambda i: (0, 0)),
            pl.BlockSpec((tn, 1), lambda i: (i, 0)),
            pl.BlockSpec((tn, 1), lambda i: (i, 0)),
            pl.BlockSpec((out, hid), lambda i: (0, 0)),
            pl.BlockSpec((1, out), lambda i: (0, 0)),
            pl.BlockSpec((tn, 1), lambda i: (i, 0)),
            pl.BlockSpec((tn, 1), lambda i: (i, 0)),
        ],
        out_specs=pl.BlockSpec((b, tn, c), lambda i: (0, i, 0)),
        out_shape=jax.ShapeDtypeStruct((b, n, c), jnp.float32),
    )(nbrp4, nbrx4, ppad3, a, w1p, g1c, be1c, w2, b2r, g2c, be2c)


def kernel(x, points, indices, W1, b1, g1, be1, W2, b2, g2, be2):
    b, n, c = x.shape
    k = indices.shape[2]
    hid = W1.shape[0]
    out = W2.shape[0]

    xflat = x.astype(jnp.bfloat16).reshape(b * n, c)
    ppad = jnp.pad(points, ((0, 0), (0, 0), (0, PW - points.shape[2])))
    ppad = ppad.at[:, :, 3].set(1.0)
    ppad = ppad.reshape(b * n, PW)
    boff = (jnp.arange(b, dtype=jnp.int32) * n)
    idx32 = indices.astype(jnp.int32)
    xidx = (idx32.transpose(0, 2, 1) + boff[:, None, None]).reshape(-1)
    pidx = (idx32.reshape(b, n, 2, k // 2).transpose(0, 2, 1, 3)
            + boff[:, None, None, None]).reshape(-1)

    w1r = W1.reshape(hid, k, points.shape[2])
    w1pad = jnp.pad(w1r, ((0, 0), (0, 0), (0, PW - points.shape[2])))
    w1p = w1pad.transpose(1, 2, 0).reshape(k * PW, hid)
    a = w1p.reshape(k, PW, hid).sum(axis=0).at[3].set(b1)
    w1pb = w1p.astype(jnp.bfloat16)
    w2b = W2.astype(jnp.bfloat16)
    b2r = b2.reshape(1, out)
    g1c = g1.reshape(n, 1)
    be1c = be1.reshape(n, 1)
    g2c = g2.reshape(n, 1)
    be2c = be2.reshape(n, 1)

    rows = b * n * k
    nbrx, nbrp = _sc_gather(xflat, ppad, xidx, pidx,
                            rows_per_worker=rows // 32)
    nbrx4 = nbrx.reshape(b, k, n, c)
    nbrp4 = nbrp.reshape(b, 2, n, c)

    tn = 400 if n % 400 == 0 else n
    res = _tc_mlp(nbrp4, nbrx4, ppad.reshape(b, n, PW), a, w1pb, g1c,
                  be1c, w2b, b2r, g2c, be2c, tn)
    return (res, points, indices)


# speedup vs baseline: 1.7248x; 1.7248x over previous
import functools

import jax
import jax.numpy as jnp
from jax import lax
from jax.experimental import pallas as pl
from jax.experimental.pallas import tpu as pltpu
from jax.experimental.pallas import tpu_sc as plsc

PW = 16


def _sc_gather(xflat, ppad, xidx, pidx, rows_per_worker):
    rows, c = xidx.shape[0], xflat.shape[1]
    nw = 32
    assert rows == nw * rows_per_worker
    chunk = 128
    nfull = rows_per_worker // chunk
    tail = rows_per_worker - nfull * chunk
    assert nfull % 2 == 0 and tail % 8 == 0 and tail < chunk

    mesh = plsc.VectorSubcoreMesh(core_axis_name="c", subcore_axis_name="s")

    @functools.partial(
        pl.kernel,
        out_type=[
            jax.ShapeDtypeStruct((rows, c), jnp.float32),
            jax.ShapeDtypeStruct((rows, PW), jnp.float32),
        ],
        mesh=mesh,
        compiler_params=pltpu.CompilerParams(use_tc_tiling_on_sc=False),
        scratch_types=[
            pltpu.VMEM((rows_per_worker,), jnp.int32),
            pltpu.VMEM((rows_per_worker,), jnp.int32),
            pltpu.VMEM((chunk, c), jnp.float32),
            pltpu.VMEM((chunk, c), jnp.float32),
            pltpu.VMEM((chunk, PW), jnp.float32),
            pltpu.VMEM((chunk, PW), jnp.float32),
            pltpu.SemaphoreType.DMA,
            pltpu.SemaphoreType.DMA,
            pltpu.SemaphoreType.DMA,
            pltpu.SemaphoreType.DMA,
        ],
    )
    def k(xflat_hbm, ppad_hbm, xidx_hbm, pidx_hbm, nbrx_hbm, nbrp_hbm,
          xidx_v, pidx_v, xr0, xr1, pr0, pr1, sx0, sx1, sp0, sp1):
        wid = lax.axis_index("s") * 2 + lax.axis_index("c")
        base = wid * rows_per_worker
        pltpu.sync_copy(xidx_hbm.at[pl.ds(base, rows_per_worker)], xidx_v)
        pltpu.sync_copy(pidx_hbm.at[pl.ds(base, rows_per_worker)], pidx_v)

        def start(g, xr, pr, sx, sp, nrows=chunk):
            off = pl.multiple_of(g * chunk, chunk)
            pltpu.async_copy(
                xflat_hbm.at[xidx_v.at[pl.ds(off, nrows)]],
                xr.at[pl.ds(0, nrows)], sx)
            pltpu.async_copy(
                ppad_hbm.at[pidx_v.at[pl.ds(off, nrows)]],
                pr.at[pl.ds(0, nrows)], sp)

        def drain(xr, pr, sx, sp, nrows=chunk):
            pltpu.make_async_copy(
                xflat_hbm.at[pl.ds(0, nrows)], xr.at[pl.ds(0, nrows)],
                sx).wait()
            pltpu.make_async_copy(
                ppad_hbm.at[pl.ds(0, nrows)], pr.at[pl.ds(0, nrows)],
                sp).wait()

        def write(g, xr, pr, nrows=chunk):
            off = pl.multiple_of(g * chunk, chunk)
            pltpu.sync_copy(xr.at[pl.ds(0, nrows)],
                            nbrx_hbm.at[pl.ds(base + off, nrows)])
            pltpu.sync_copy(pr.at[pl.ds(0, nrows)],
                            nbrp_hbm.at[pl.ds(base + off, nrows)])

        start(0, xr0, pr0, sx0, sp0)

        def body(go, carry):
            g0 = pl.multiple_of(go * 2, 2)
            start(g0 + 1, xr1, pr1, sx1, sp1)
            drain(xr0, pr0, sx0, sp0)
            write(g0, xr0, pr0)
            start(g0 + 2, xr0, pr0, sx0, sp0)
            drain(xr1, pr1, sx1, sp1)
            write(g0 + 1, xr1, pr1)
            return carry

        lax.fori_loop(0, nfull // 2 - 1, body, 0)
        g0 = nfull - 2
        start(g0 + 1, xr1, pr1, sx1, sp1)
        drain(xr0, pr0, sx0, sp0)
        write(g0, xr0, pr0)
        if tail:
            start(nfull, xr0, pr0, sx0, sp0, nrows=tail)
        drain(xr1, pr1, sx1, sp1)
        write(g0 + 1, xr1, pr1)
        if tail:
            drain(xr0, pr0, sx0, sp0, nrows=tail)
            write(nfull, xr0, pr0, nrows=tail)

    return k(xflat, ppad, xidx, pidx)


def _tc_body(nbrp_ref, nbrx_ref, pp_ref, a_ref, w1p_ref, g1_ref,
             be1_ref, w2_ref, b2_ref, g2_ref, be2_ref, out_ref):
    b, kk, tn, c = nbrx_ref.shape
    hid = w1p_ref.shape[1]
    out = w2_ref.shape[0]

    pp = pp_ref[...].reshape(b * tn, PW)
    hi = nbrp_ref[:, 0].reshape(b * tn, c).astype(jnp.bfloat16)
    lo = nbrp_ref[:, 1].reshape(b * tn, c).astype(jnp.bfloat16)
    h = (jnp.dot(pp, a_ref[...], preferred_element_type=jnp.float32)
         - (jnp.dot(hi, w1p_ref[0:c], preferred_element_type=jnp.float32)
            + jnp.dot(lo, w1p_ref[c:2 * c],
                      preferred_element_type=jnp.float32)))
    h3 = h.reshape(b, tn, hid)
    m1 = jnp.mean(h3, axis=(0, 2), keepdims=True)
    q1 = jnp.mean(h3 * h3, axis=(0, 2), keepdims=True)
    rs1 = lax.rsqrt(q1 - m1 * m1 + 1e-5)
    sc1 = rs1 * g1_ref[...][None]
    sh1 = be1_ref[...][None] - m1 * sc1
    hr = jnp.maximum(h3 * sc1 + sh1, 0.0)
    hrb = hr.astype(jnp.bfloat16).reshape(b * tn, hid)

    o = lax.dot_general(hrb, w2_ref[...], (((1,), (1,)), ((), ())),
                        preferred_element_type=jnp.float32) + b2_ref[...]
    o3 = o.reshape(b, tn, out)
    m2 = jnp.mean(o3, axis=(0, 2), keepdims=True)
    q2 = jnp.mean(o3 * o3, axis=(0, 2), keepdims=True)
    rs2 = lax.rsqrt(q2 - m2 * m2 + 1e-5)
    sc2 = rs2 * g2_ref[...][None]
    sh2 = be2_ref[...][None] - m2 * sc2

    acc = jnp.zeros((b, tn, c), jnp.float32)
    for j in range(kk):
        yj = jnp.maximum(o3[:, :, j * c:(j + 1) * c] * sc2 + sh2, 0.0)
        acc = acc + yj * nbrx_ref[:, j].astype(jnp.float32)
    out_ref[...] = acc


def _tc_mlp(nbrp4, nbrx4, ppad3, a, w1p, g1c, be1c, w2, b2r, g2c, be2c, tn):
    b, kk, n, c = nbrx4.shape
    hid = w1p.shape[1]
    out = w2.shape[0]
    grid = (n // tn,)
    return pl.pallas_call(
        _tc_body,
        grid=grid,
        in_specs=[
            pl.BlockSpec((b, 2, tn, c), lambda i: (0, 0, i, 0)),
            pl.BlockSpec((b, kk, tn, c), lambda i: (0, 0, i, 0)),
            pl.BlockSpec((b, tn, PW), lambda i: (0, i, 0)),
            pl.BlockSpec((PW, hid), lambda i: (0, 0)),
            pl.BlockSpec((2 * c, hid), lambda i: (0, 0)),
            pl.BlockSpec((tn, 1), lambda i: (i, 0)),
            pl.BlockSpec((tn, 1), lambda i: (i, 0)),
            pl.BlockSpec((out, hid), lambda i: (0, 0)),
            pl.BlockSpec((1, out), lambda i: (0, 0)),
            pl.BlockSpec((tn, 1), lambda i: (i, 0)),
            pl.BlockSpec((tn, 1), lambda i: (i, 0)),
        ],
        out_specs=pl.BlockSpec((b, tn, c), lambda i: (0, i, 0)),
        out_shape=jax.ShapeDtypeStruct((b, n, c), jnp.float32),
    )(nbrp4, nbrx4, ppad3, a, w1p, g1c, be1c, w2, b2r, g2c, be2c)


def kernel(x, points, indices, W1, b1, g1, be1, W2, b2, g2, be2):
    b, n, c = x.shape
    k = indices.shape[2]
    hid = W1.shape[0]
    out = W2.shape[0]

    xflat = x.reshape(b * n, c)
    ppad = jnp.pad(points, ((0, 0), (0, 0), (0, PW - points.shape[2])))
    ppad = ppad.at[:, :, 3].set(1.0)
    ppad = ppad.reshape(b * n, PW)
    boff = (jnp.arange(b, dtype=jnp.int32) * n)
    idx32 = indices.astype(jnp.int32)
    xidx = (idx32.transpose(0, 2, 1) + boff[:, None, None]).reshape(-1)
    pidx = (idx32.reshape(b, n, 2, k // 2).transpose(0, 2, 1, 3)
            + boff[:, None, None, None]).reshape(-1)

    w1r = W1.reshape(hid, k, points.shape[2])
    w1pad = jnp.pad(w1r, ((0, 0), (0, 0), (0, PW - points.shape[2])))
    w1p = w1pad.transpose(1, 2, 0).reshape(k * PW, hid)
    a = w1p.reshape(k, PW, hid).sum(axis=0).at[3].set(b1)
    w1pb = w1p.astype(jnp.bfloat16)
    w2b = W2.astype(jnp.bfloat16)
    b2r = b2.reshape(1, out)
    g1c = g1.reshape(n, 1)
    be1c = be1.reshape(n, 1)
    g2c = g2.reshape(n, 1)
    be2c = be2.reshape(n, 1)

    rows = b * n * k
    nbrx, nbrp = _sc_gather(xflat, ppad, xidx, pidx,
                            rows_per_worker=rows // 32)
    nbrx4 = nbrx.reshape(b, k, n, c)
    nbrp4 = nbrp.reshape(b, 2, n, c)

    tn = 400 if n % 400 == 0 else n
    res = _tc_mlp(nbrp4, nbrx4, ppad.reshape(b, n, PW), a, w1pb, g1c,
                  be1c, w2b, b2r, g2c, be2c, tn)
    return (res, points, indices)


# speedup vs baseline: 1.9192x; 1.1127x over previous
import functools

import jax
import jax.numpy as jnp
from jax import lax
from jax.experimental import pallas as pl
from jax.experimental.pallas import tpu as pltpu
from jax.experimental.pallas import tpu_sc as plsc

PW = 16


def _sc_gather(xflat, ppad, xidx, pidx, rows_per_worker):
    rows, c = xidx.shape[0], xflat.shape[1]
    nw = 32
    assert rows == nw * rows_per_worker
    chunk = 128
    nfull = rows_per_worker // chunk
    tail = rows_per_worker - nfull * chunk
    assert nfull % 2 == 0 and tail % 8 == 0 and tail < chunk

    mesh = plsc.VectorSubcoreMesh(core_axis_name="c", subcore_axis_name="s")

    @functools.partial(
        pl.kernel,
        out_type=[
            jax.ShapeDtypeStruct((rows, c), jnp.float32),
            jax.ShapeDtypeStruct((rows, PW), jnp.float32),
        ],
        mesh=mesh,
        compiler_params=pltpu.CompilerParams(use_tc_tiling_on_sc=False),
        scratch_types=[
            pltpu.VMEM((rows_per_worker,), jnp.int32),
            pltpu.VMEM((rows_per_worker,), jnp.int32),
            pltpu.VMEM((chunk, c), jnp.float32),
            pltpu.VMEM((chunk, c), jnp.float32),
            pltpu.VMEM((chunk, PW), jnp.float32),
            pltpu.VMEM((chunk, PW), jnp.float32),
            pltpu.SemaphoreType.DMA,
            pltpu.SemaphoreType.DMA,
            pltpu.SemaphoreType.DMA,
            pltpu.SemaphoreType.DMA,
        ],
    )
    def k(xflat_hbm, ppad_hbm, xidx_hbm, pidx_hbm, nbrx_hbm, nbrp_hbm,
          xidx_v, pidx_v, xr0, xr1, pr0, pr1, sx0, sx1, sp0, sp1):
        wid = lax.axis_index("s") * 2 + lax.axis_index("c")
        base = wid * rows_per_worker
        pltpu.sync_copy(xidx_hbm.at[pl.ds(base, rows_per_worker)], xidx_v)
        pltpu.sync_copy(pidx_hbm.at[pl.ds(base, rows_per_worker)], pidx_v)

        def start(g, xr, pr, sx, sp, nrows=chunk):
            off = pl.multiple_of(g * chunk, chunk)
            pltpu.async_copy(
                xflat_hbm.at[xidx_v.at[pl.ds(off, nrows)]],
                xr.at[pl.ds(0, nrows)], sx)
            pltpu.async_copy(
                ppad_hbm.at[pidx_v.at[pl.ds(off, nrows)]],
                pr.at[pl.ds(0, nrows)], sp)

        def drain(xr, pr, sx, sp, nrows=chunk):
            pltpu.make_async_copy(
                xflat_hbm.at[pl.ds(0, nrows)], xr.at[pl.ds(0, nrows)],
                sx).wait()
            pltpu.make_async_copy(
                ppad_hbm.at[pl.ds(0, nrows)], pr.at[pl.ds(0, nrows)],
                sp).wait()

        def write(g, xr, pr, nrows=chunk):
            off = pl.multiple_of(g * chunk, chunk)
            pltpu.sync_copy(xr.at[pl.ds(0, nrows)],
                            nbrx_hbm.at[pl.ds(base + off, nrows)])
            pltpu.sync_copy(pr.at[pl.ds(0, nrows)],
                            nbrp_hbm.at[pl.ds(base + off, nrows)])

        start(0, xr0, pr0, sx0, sp0)

        def body(go, carry):
            g0 = pl.multiple_of(go * 2, 2)
            start(g0 + 1, xr1, pr1, sx1, sp1)
            drain(xr0, pr0, sx0, sp0)
            write(g0, xr0, pr0)
            start(g0 + 2, xr0, pr0, sx0, sp0)
            drain(xr1, pr1, sx1, sp1)
            write(g0 + 1, xr1, pr1)
            return carry

        lax.fori_loop(0, nfull // 2 - 1, body, 0)
        g0 = nfull - 2
        start(g0 + 1, xr1, pr1, sx1, sp1)
        drain(xr0, pr0, sx0, sp0)
        write(g0, xr0, pr0)
        if tail:
            start(nfull, xr0, pr0, sx0, sp0, nrows=tail)
        drain(xr1, pr1, sx1, sp1)
        write(g0 + 1, xr1, pr1)
        if tail:
            drain(xr0, pr0, sx0, sp0, nrows=tail)
            write(nfull, xr0, pr0, nrows=tail)

    return k(xflat, ppad, xidx, pidx)


def _tc_body(nbrp_ref, nbrx_ref, pp_ref, a_ref, w1p_ref, g1_ref,
             be1_ref, w2_ref, b2_ref, g2_ref, be2_ref, out_ref):
    b, kk, tn, c = nbrx_ref.shape
    hid = w1p_ref.shape[1]
    out = w2_ref.shape[0]

    pp = pp_ref[...].reshape(b * tn, PW)
    hi = nbrp_ref[:, 0].reshape(b * tn, c).astype(jnp.bfloat16)
    lo = nbrp_ref[:, 1].reshape(b * tn, c).astype(jnp.bfloat16)
    h = (jnp.dot(pp, a_ref[...], preferred_element_type=jnp.float32)
         - (jnp.dot(hi, w1p_ref[0:c], preferred_element_type=jnp.float32)
            + jnp.dot(lo, w1p_ref[c:2 * c],
                      preferred_element_type=jnp.float32)))
    h3 = h.reshape(b, tn, hid)
    m1 = jnp.mean(h3, axis=(0, 2), keepdims=True)
    q1 = jnp.mean(h3 * h3, axis=(0, 2), keepdims=True)
    rs1 = lax.rsqrt(q1 - m1 * m1 + 1e-5)
    sc1 = rs1 * g1_ref[...][None]
    sh1 = be1_ref[...][None] - m1 * sc1
    hr = jnp.maximum(h3 * sc1 + sh1, 0.0)
    hrb = hr.astype(jnp.bfloat16).reshape(b * tn, hid)

    o = lax.dot_general(hrb, w2_ref[...], (((1,), (1,)), ((), ())),
                        preferred_element_type=jnp.float32) + b2_ref[...]
    o3 = o.reshape(b, tn, out)
    m2 = jnp.mean(o3, axis=(0, 2), keepdims=True)
    q2 = jnp.mean(o3 * o3, axis=(0, 2), keepdims=True)
    rs2 = lax.rsqrt(q2 - m2 * m2 + 1e-5)
    sc2 = rs2 * g2_ref[...][None]
    sh2 = be2_ref[...][None] - m2 * sc2

    acc = jnp.zeros((b, tn, c), jnp.float32)
    for j in range(kk):
        yj = jnp.maximum(o3[:, :, j * c:(j + 1) * c] * sc2 + sh2, 0.0)
        acc = acc + yj * nbrx_ref[:, j].astype(jnp.float32)
    out_ref[...] = acc


def _tc_mlp(nbrp4, nbrx4, ppad3, a, w1p, g1c, be1c, w2, b2r, g2c, be2c, tn):
    b, kk, n, c = nbrx4.shape
    hid = w1p.shape[1]
    out = w2.shape[0]
    grid = (n // tn,)
    return pl.pallas_call(
        _tc_body,
        grid=grid,
        in_specs=[
            pl.BlockSpec((b, 2, tn, c), lambda i: (0, 0, i, 0)),
            pl.BlockSpec((b, kk, tn, c), lambda i: (0, 0, i, 0)),
            pl.BlockSpec((b, tn, PW), lambda i: (0, i, 0)),
            pl.BlockSpec((PW, hid), lambda i: (0, 0)),
            pl.BlockSpec((2 * c, hid), lambda i: (0, 0)),
            pl.BlockSpec((tn, 1), lambda i: (i, 0)),
            pl.BlockSpec((tn, 1), lambda i: (i, 0)),
            pl.BlockSpec((out, hid), lambda i: (0, 0)),
            pl.BlockSpec((1, out), lambda i: (0, 0)),
            pl.BlockSpec((tn, 1), lambda i: (i, 0)),
            pl.BlockSpec((tn, 1), lambda i: (i, 0)),
        ],
        out_specs=pl.BlockSpec((b, tn, c), lambda i: (0, i, 0)),
        out_shape=jax.ShapeDtypeStruct((b, n, c), jnp.float32),
    )(nbrp4, nbrx4, ppad3, a, w1p, g1c, be1c, w2, b2r, g2c, be2c)


def kernel(x, points, indices, W1, b1, g1, be1, W2, b2, g2, be2):
    b, n, c = x.shape
    k = indices.shape[2]
    hid = W1.shape[0]
    out = W2.shape[0]

    xflat = x.reshape(b * n, c)
    ppad = jnp.concatenate(
        [points, jnp.ones((b, n, 1), points.dtype),
         jnp.zeros((b, n, PW - 4), points.dtype)], axis=-1).reshape(b * n, PW)
    boff = (jnp.arange(b, dtype=jnp.int32) * n)
    idx32 = indices.astype(jnp.int32)
    xidx = (idx32.transpose(0, 2, 1) + boff[:, None, None]).reshape(-1)
    kh = k // 2
    pidx = (jnp.stack([idx32[:, :, 0:kh], idx32[:, :, kh:k]], axis=1)
            + boff[:, None, None, None]).reshape(-1)

    w1r = W1.reshape(hid, k, points.shape[2])
    w1pad = jnp.pad(w1r, ((0, 0), (0, 0), (0, PW - points.shape[2])))
    w1p = w1pad.transpose(1, 2, 0).reshape(k * PW, hid)
    a = w1p.reshape(k, PW, hid).sum(axis=0).at[3].set(b1)
    w1pb = w1p.astype(jnp.bfloat16)
    w2b = W2.astype(jnp.bfloat16)
    b2r = b2.reshape(1, out)
    g1c = g1.reshape(n, 1)
    be1c = be1.reshape(n, 1)
    g2c = g2.reshape(n, 1)
    be2c = be2.reshape(n, 1)

    rows = b * n * k
    nbrx, nbrp = _sc_gather(xflat, ppad, xidx, pidx,
                            rows_per_worker=rows // 32)
    nbrx4 = nbrx.reshape(b, k, n, c)
    nbrp4 = nbrp.reshape(b, 2, n, c)

    tn = 400 if n % 400 == 0 else n
    res = _tc_mlp(nbrp4, nbrx4, ppad.reshape(b, n, PW), a, w1pb, g1c,
                  be1c, w2b, b2r, g2c, be2c, tn)
    return (res, points, indices)


# speedup vs baseline: 2.0038x; 1.0441x over previous
import functools

import jax
import jax.numpy as jnp
from jax import lax
from jax.experimental import pallas as pl
from jax.experimental.pallas import tpu as pltpu
from jax.experimental.pallas import tpu_sc as plsc

PW = 16


def _sc_gather(xflat, ppad, xidx, pidx, rows_per_worker, chunk):
    rows, c = xidx.shape[0], xflat.shape[1]
    nw = 32
    assert rows == nw * rows_per_worker
    nfull = rows_per_worker // chunk
    tail = rows_per_worker - nfull * chunk
    assert chunk <= 128 and chunk % 8 == 0
    assert nfull % 2 == 0 and tail % 8 == 0 and tail < chunk

    mesh = plsc.VectorSubcoreMesh(core_axis_name="c", subcore_axis_name="s")

    @functools.partial(
        pl.kernel,
        out_type=[
            jax.ShapeDtypeStruct((rows, c), jnp.float32),
            jax.ShapeDtypeStruct((rows, PW), jnp.float32),
        ],
        mesh=mesh,
        compiler_params=pltpu.CompilerParams(use_tc_tiling_on_sc=False),
        scratch_types=[
            pltpu.VMEM((rows_per_worker,), jnp.int32),
            pltpu.VMEM((rows_per_worker,), jnp.int32),
            pltpu.VMEM((chunk, c), jnp.float32),
            pltpu.VMEM((chunk, c), jnp.float32),
            pltpu.VMEM((chunk, PW), jnp.float32),
            pltpu.VMEM((chunk, PW), jnp.float32),
            pltpu.SemaphoreType.DMA,
            pltpu.SemaphoreType.DMA,
            pltpu.SemaphoreType.DMA,
            pltpu.SemaphoreType.DMA,
        ],
    )
    def k(xflat_hbm, ppad_hbm, xidx_hbm, pidx_hbm, nbrx_hbm, nbrp_hbm,
          xidx_v, pidx_v, xr0, xr1, pr0, pr1, sx0, sx1, sp0, sp1):
        wid = lax.axis_index("s") * 2 + lax.axis_index("c")
        base = wid * rows_per_worker
        pltpu.sync_copy(xidx_hbm.at[pl.ds(base, rows_per_worker)], xidx_v)
        pltpu.sync_copy(pidx_hbm.at[pl.ds(base, rows_per_worker)], pidx_v)

        def start(g, xr, pr, sx, sp, nrows=chunk):
            off = pl.multiple_of(g * chunk, chunk)
            pltpu.async_copy(
                xflat_hbm.at[xidx_v.at[pl.ds(off, nrows)]],
                xr.at[pl.ds(0, nrows)], sx)
            pltpu.async_copy(
                ppad_hbm.at[pidx_v.at[pl.ds(off, nrows)]],
                pr.at[pl.ds(0, nrows)], sp)

        def drain(xr, pr, sx, sp, nrows=chunk):
            pltpu.make_async_copy(
                xflat_hbm.at[pl.ds(0, nrows)], xr.at[pl.ds(0, nrows)],
                sx).wait()
            pltpu.make_async_copy(
                ppad_hbm.at[pl.ds(0, nrows)], pr.at[pl.ds(0, nrows)],
                sp).wait()

        def write(g, xr, pr, nrows=chunk):
            off = pl.multiple_of(g * chunk, chunk)
            pltpu.sync_copy(xr.at[pl.ds(0, nrows)],
                            nbrx_hbm.at[pl.ds(base + off, nrows)])
            pltpu.sync_copy(pr.at[pl.ds(0, nrows)],
                            nbrp_hbm.at[pl.ds(base + off, nrows)])

        start(0, xr0, pr0, sx0, sp0)

        def body(go, carry):
            g0 = pl.multiple_of(go * 2, 2)
            start(g0 + 1, xr1, pr1, sx1, sp1)
            drain(xr0, pr0, sx0, sp0)
            write(g0, xr0, pr0)
            start(g0 + 2, xr0, pr0, sx0, sp0)
            drain(xr1, pr1, sx1, sp1)
            write(g0 + 1, xr1, pr1)
            return carry

        lax.fori_loop(0, nfull // 2 - 1, body, 0)
        g0 = nfull - 2
        start(g0 + 1, xr1, pr1, sx1, sp1)
        drain(xr0, pr0, sx0, sp0)
        write(g0, xr0, pr0)
        if tail:
            start(nfull, xr0, pr0, sx0, sp0, nrows=tail)
        drain(xr1, pr1, sx1, sp1)
        write(g0 + 1, xr1, pr1)
        if tail:
            drain(xr0, pr0, sx0, sp0, nrows=tail)
            write(nfull, xr0, pr0, nrows=tail)

    return k(xflat, ppad, xidx, pidx)


def _tc_body(nbrp_ref, nbrx_ref, pp_ref, a_ref, w1p_ref, g1_ref,
             be1_ref, w2_ref, b2_ref, g2_ref, be2_ref, out_ref):
    b, kk, tn, c = nbrx_ref.shape
    hid = w1p_ref.shape[1]
    out = w2_ref.shape[0]

    pp = pp_ref[...].reshape(b * tn, PW)
    hi = nbrp_ref[:, 0].reshape(b * tn, c).astype(jnp.bfloat16)
    lo = nbrp_ref[:, 1].reshape(b * tn, c).astype(jnp.bfloat16)
    h = (jnp.dot(pp, a_ref[...], preferred_element_type=jnp.float32)
         - (jnp.dot(hi, w1p_ref[0:c], preferred_element_type=jnp.float32)
            + jnp.dot(lo, w1p_ref[c:2 * c],
                      preferred_element_type=jnp.float32)))
    h3 = h.reshape(b, tn, hid)
    m1 = jnp.mean(h3, axis=(0, 2), keepdims=True)
    q1 = jnp.mean(h3 * h3, axis=(0, 2), keepdims=True)
    rs1 = lax.rsqrt(q1 - m1 * m1 + 1e-5)
    sc1 = rs1 * g1_ref[...][None]
    sh1 = be1_ref[...][None] - m1 * sc1
    hr = jnp.maximum(h3 * sc1 + sh1, 0.0)
    hrb = hr.astype(jnp.bfloat16).reshape(b * tn, hid)

    o = lax.dot_general(hrb, w2_ref[...], (((1,), (1,)), ((), ())),
                        preferred_element_type=jnp.float32) + b2_ref[...]
    o3 = o.reshape(b, tn, out)
    m2 = jnp.mean(o3, axis=(0, 2), keepdims=True)
    q2 = jnp.mean(o3 * o3, axis=(0, 2), keepdims=True)
    rs2 = lax.rsqrt(q2 - m2 * m2 + 1e-5)
    sc2 = rs2 * g2_ref[...][None]
    sh2 = be2_ref[...][None] - m2 * sc2

    acc = jnp.zeros((b, tn, c), jnp.float32)
    for j in range(kk):
        yj = jnp.maximum(o3[:, :, j * c:(j + 1) * c] * sc2 + sh2, 0.0)
        acc = acc + yj * nbrx_ref[:, j].astype(jnp.float32)
    out_ref[...] = acc


def _tc_mlp(nbrp4, nbrx4, ppad3, a, w1p, g1c, be1c, w2, b2r, g2c, be2c, tn):
    b, kk, n, c = nbrx4.shape
    hid = w1p.shape[1]
    out = w2.shape[0]
    grid = (n // tn,)
    return pl.pallas_call(
        _tc_body,
        grid=grid,
        in_specs=[
            pl.BlockSpec((b, 2, tn, c), lambda i: (0, 0, i, 0)),
            pl.BlockSpec((b, kk, tn, c), lambda i: (0, 0, i, 0)),
            pl.BlockSpec((b, tn, PW), lambda i: (0, i, 0)),
            pl.BlockSpec((PW, hid), lambda i: (0, 0)),
            pl.BlockSpec((2 * c, hid), lambda i: (0, 0)),
            pl.BlockSpec((tn, 1), lambda i: (i, 0)),
            pl.BlockSpec((tn, 1), lambda i: (i, 0)),
            pl.BlockSpec((out, hid), lambda i: (0, 0)),
            pl.BlockSpec((1, out), lambda i: (0, 0)),
            pl.BlockSpec((tn, 1), lambda i: (i, 0)),
            pl.BlockSpec((tn, 1), lambda i: (i, 0)),
        ],
        out_specs=pl.BlockSpec((b, tn, c), lambda i: (0, i, 0)),
        out_shape=jax.ShapeDtypeStruct((b, n, c), jnp.float32),
    )(nbrp4, nbrx4, ppad3, a, w1p, g1c, be1c, w2, b2r, g2c, be2c)


def kernel(x, points, indices, W1, b1, g1, be1, W2, b2, g2, be2):
    b, n, c = x.shape
    k = indices.shape[2]
    hid = W1.shape[0]
    out = W2.shape[0]

    xflat = x.reshape(b * n, c)
    ppad = jnp.concatenate(
        [points, jnp.ones((b, n, 1), points.dtype),
         jnp.zeros((b, n, PW - 4), points.dtype)], axis=-1).reshape(b * n, PW)
    boff = (jnp.arange(b, dtype=jnp.int32) * n)
    idx32 = indices.astype(jnp.int32)
    kh = k // 2

    w1r = W1.reshape(hid, k, points.shape[2])
    w1pad = jnp.pad(w1r, ((0, 0), (0, 0), (0, PW - points.shape[2])))
    w1p = w1pad.transpose(1, 2, 0).reshape(k * PW, hid)
    a = w1p.reshape(k, PW, hid).sum(axis=0).at[3].set(b1)
    w1pb = w1p.astype(jnp.bfloat16)
    w2b = W2.astype(jnp.bfloat16)
    b2r = b2.reshape(1, out)
    ppad3 = ppad.reshape(b, n, PW)

    if n == 10000:
        splits, chunks, tn = (4800, 5200), (96, 104), 400
    else:
        splits, chunks, tn = (n,), (None,), n
    outs = []
    n0 = 0
    for na, chunk in zip(splits, chunks):
        sl = slice(n0, n0 + na)
        idx_h = idx32[:, sl]
        xidx = (idx_h.transpose(0, 2, 1) + boff[:, None, None]).reshape(-1)
        pidx = (jnp.stack([idx_h[:, :, 0:kh], idx_h[:, :, kh:k]], axis=1)
                + boff[:, None, None, None]).reshape(-1)
        rows = b * na * k
        if chunk is None:
            chunk = 128
        nbrx, nbrp = _sc_gather(xflat, ppad, xidx, pidx,
                                rows_per_worker=rows // 32, chunk=chunk)
        nbrx4 = nbrx.reshape(b, k, na, c)
        nbrp4 = nbrp.reshape(b, 2, na, c)
        outs.append(_tc_mlp(nbrp4, nbrx4, ppad3[:, sl], a, w1pb,
                            g1[sl].reshape(na, 1), be1[sl].reshape(na, 1),
                            w2b, b2r, g2[sl].reshape(na, 1),
                            be2[sl].reshape(na, 1), tn))
        n0 += na
    res = outs[0] if len(outs) == 1 else jnp.concatenate(outs, axis=1)
    return (res, points, indices)
